# baseline (speedup 1.0000x reference)
import functools

import jax
import jax.numpy as jnp
from jax import lax
from jax.experimental import pallas as pl
from jax.experimental.pallas import tpu as pltpu
from jax.experimental.pallas import tpu_sc as plsc

EMBED = 128
HIDDEN = 64
B = 1024
T = 200

_SEGS = (8, 64, 64, 56, 8)

_NC = 2
_NS = 16
_NW = _NC * _NS
_CHUNK = 64


def _sc_gather(table, idx3, rows, nchunk):
    rpw = rows // _NW
    mesh = plsc.VectorSubcoreMesh(core_axis_name="c", subcore_axis_name="s")

    @functools.partial(
        pl.kernel,
        mesh=mesh,
        out_type=jax.ShapeDtypeStruct((rows, EMBED), jnp.float32),
        scratch_types=[
            pltpu.VMEM((nchunk, _CHUNK), jnp.int32),
            pltpu.VMEM((_CHUNK, EMBED), jnp.float32),
            pltpu.VMEM((_CHUNK, EMBED), jnp.float32),
            pltpu.VMEM((_CHUNK, EMBED), jnp.float32),
            pltpu.VMEM((_CHUNK, EMBED), jnp.float32),
            pltpu.SemaphoreType.DMA,
            pltpu.SemaphoreType.DMA,
            pltpu.SemaphoreType.DMA,
            pltpu.SemaphoreType.DMA,
            pltpu.SemaphoreType.DMA,
            pltpu.SemaphoreType.DMA,
            pltpu.SemaphoreType.DMA,
            pltpu.SemaphoreType.DMA,
        ],
    )
    def gather_kernel(table_hbm, idx_hbm, out_hbm, idx_v,
                      r0, r1, r2, r3, g0, g1, g2, g3, w0, w1, w2, w3):
        bufs = (r0, r1, r2, r3)
        gsems = (g0, g1, g2, g3)
        wsems = (w0, w1, w2, w3)
        wid = lax.axis_index("s") * _NC + lax.axis_index("c")
        base = wid * rpw
        pltpu.sync_copy(idx_hbm.at[wid], idx_v)

        def start(i, b):
            pltpu.async_copy(table_hbm.at[idx_v.at[i]], bufs[b], gsems[b])

        def gdrain(b):
            pltpu.make_async_copy(table_hbm.at[idx_v.at[0]], bufs[b], gsems[b]).wait()

        def wb(i, b):
            off = pl.multiple_of(base + i * _CHUNK, _CHUNK)
            pltpu.async_copy(bufs[b], out_hbm.at[pl.ds(off, _CHUNK)], wsems[b])

        def wdrain(b):
            pltpu.make_async_copy(bufs[b], out_hbm.at[pl.ds(0, _CHUNK)], wsems[b]).wait()

        for b in range(4):
            start(b, b)

        def body(g, carry):
            for u in range(4):
                i = g * 4 + u
                gdrain(u)
                wb(i, u)
                nxt = i + 2
                b2 = (u + 2) % 4

                @pl.when((nxt >= 4) & (nxt < nchunk))
                def _refill():
                    wdrain(b2)
                    start(nxt, b2)

            return carry

        lax.fori_loop(0, nchunk // 4, body, 0)
        for b in range(4):
            wdrain(b)

    return gather_kernel(table, idx3)


_NT = 8


def _make_gru_body(tseg):
    def _gru_body(e_ref, h0_ref, wih_ref, whh_ref, bih_ref, bhh_ref, fcw_ref,
                  fcb_ref, hout_ref, y_ref, h_scr):
        t = pl.program_id(0)

        @pl.when(t == 0)
        def _init():
            h_scr[...] = h0_ref[...]

        h = h_scr[...]
        for j in range(_NT):
            e = e_ref[j]
            gi = lax.dot_general(wih_ref[...], e, (((1,), (1,)), ((), ())),
                                 preferred_element_type=jnp.float32) + bih_ref[...]
            gh = jnp.dot(whh_ref[...], h, preferred_element_type=jnp.float32) + bhh_ref[...]
            r = jax.nn.sigmoid(gi[:HIDDEN] + gh[:HIDDEN])
            z = jax.nn.sigmoid(gi[HIDDEN:2 * HIDDEN] + gh[HIDDEN:2 * HIDDEN])
            n = jnp.tanh(gi[2 * HIDDEN:] + r * gh[2 * HIDDEN:])
            h = (1.0 - z) * n + z * h
        h_scr[...] = h

        @pl.when(t == tseg // _NT - 1)
        def _finish():
            hout_ref[...] = h
            logits = jnp.sum(h * fcw_ref[...], axis=0, keepdims=True) + fcb_ref[...]
            y_ref[...] = jax.nn.sigmoid(logits)

    return _gru_body


def _tc_gru_seg(e3, h0, wih, whh, bih2, bhh2, fcwT, fcb2, tseg):
    return pl.pallas_call(
        _make_gru_body(tseg),
        grid=(tseg // _NT,),
        in_specs=[
            pl.BlockSpec((_NT, B, EMBED), lambda t: (t, 0, 0)),
            pl.BlockSpec((HIDDEN, B), lambda t: (0, 0)),
            pl.BlockSpec((3 * HIDDEN, EMBED), lambda t: (0, 0)),
            pl.BlockSpec((3 * HIDDEN, HIDDEN), lambda t: (0, 0)),
            pl.BlockSpec((3 * HIDDEN, 1), lambda t: (0, 0)),
            pl.BlockSpec((3 * HIDDEN, 1), lambda t: (0, 0)),
            pl.BlockSpec((HIDDEN, 1), lambda t: (0, 0)),
            pl.BlockSpec((1, 1), lambda t: (0, 0)),
        ],
        out_specs=[
            pl.BlockSpec((HIDDEN, B), lambda t: (0, 0)),
            pl.BlockSpec((1, B), lambda t: (0, 0)),
        ],
        out_shape=[
            jax.ShapeDtypeStruct((HIDDEN, B), jnp.float32),
            jax.ShapeDtypeStruct((1, B), jnp.float32),
        ],
        scratch_shapes=[pltpu.VMEM((HIDDEN, B), jnp.float32)],
    )(e3, h0, wih, whh, bih2, bhh2, fcwT, fcb2)


def kernel(x, emb, W_ih, W_hh, b_ih, b_hh, fc_w, fc_b):
    idx = x.astype(jnp.int32).T.reshape(-1)

    bih2 = b_ih.reshape(-1, 1)
    bhh2 = b_hh.reshape(-1, 1)
    fcwT = fc_w.reshape(-1, 1)
    fcb2 = fc_b.reshape(1, 1)

    h = jnp.zeros((HIDDEN, B), jnp.float32)
    y = None
    off = 0
    for tseg in _SEGS:
        seg_rows = tseg * B
        nchunk = seg_rows // (_NW * _CHUNK)
        idx3 = lax.dynamic_slice(idx, (off * B,), (seg_rows,)).reshape(
            _NW, nchunk, _CHUNK)
        e = _sc_gather(emb, idx3, seg_rows, nchunk)
        e3 = e.reshape(tseg, B, EMBED)
        h, y = _tc_gru_seg(e3, h, W_ih, W_hh, bih2, bhh2, fcwT, fcb2, tseg)
        off += tseg
    return y.reshape(B, 1)


# speedup vs baseline: 1.0249x; 1.0249x over previous
import functools

import jax
import jax.numpy as jnp
from jax import lax
from jax.experimental import pallas as pl
from jax.experimental.pallas import tpu as pltpu
from jax.experimental.pallas import tpu_sc as plsc

EMBED = 128
HIDDEN = 64
B = 1024
T = 200

_SEGS = (16, 64, 64, 48, 8)

_NC = 2
_NS = 16
_NW = _NC * _NS
_CHUNK = 128


def _sc_gather(table, idx3, rows, nchunk):
    rpw = rows // _NW
    mesh = plsc.VectorSubcoreMesh(core_axis_name="c", subcore_axis_name="s")

    @functools.partial(
        pl.kernel,
        mesh=mesh,
        out_type=jax.ShapeDtypeStruct((rows, EMBED), jnp.float32),
        scratch_types=[
            pltpu.VMEM((nchunk, _CHUNK), jnp.int32),
            pltpu.VMEM((_CHUNK, EMBED), jnp.float32),
            pltpu.VMEM((_CHUNK, EMBED), jnp.float32),
            pltpu.VMEM((_CHUNK, EMBED), jnp.float32),
            pltpu.VMEM((_CHUNK, EMBED), jnp.float32),
            pltpu.SemaphoreType.DMA,
            pltpu.SemaphoreType.DMA,
            pltpu.SemaphoreType.DMA,
            pltpu.SemaphoreType.DMA,
            pltpu.SemaphoreType.DMA,
            pltpu.SemaphoreType.DMA,
            pltpu.SemaphoreType.DMA,
            pltpu.SemaphoreType.DMA,
        ],
    )
    def gather_kernel(table_hbm, idx_hbm, out_hbm, idx_v,
                      r0, r1, r2, r3, g0, g1, g2, g3, w0, w1, w2, w3):
        bufs = (r0, r1, r2, r3)
        gsems = (g0, g1, g2, g3)
        wsems = (w0, w1, w2, w3)
        wid = lax.axis_index("s") * _NC + lax.axis_index("c")
        base = wid * rpw
        pltpu.sync_copy(idx_hbm.at[wid], idx_v)

        def start(i, b):
            pltpu.async_copy(table_hbm.at[idx_v.at[i]], bufs[b], gsems[b])

        def gdrain(b):
            pltpu.make_async_copy(table_hbm.at[idx_v.at[0]], bufs[b], gsems[b]).wait()

        def wb(i, b):
            off = pl.multiple_of(base + i * _CHUNK, _CHUNK)
            pltpu.async_copy(bufs[b], out_hbm.at[pl.ds(off, _CHUNK)], wsems[b])

        def wdrain(b):
            pltpu.make_async_copy(bufs[b], out_hbm.at[pl.ds(0, _CHUNK)], wsems[b]).wait()

        if nchunk < 4:
            for i in range(nchunk):
                start(i, i % 4)
            for i in range(nchunk):
                gdrain(i % 4)
                wb(i, i % 4)
            for i in range(nchunk):
                wdrain(i % 4)
            return

        for b in range(4):
            start(b, b)

        def body(g, carry):
            for u in range(4):
                i = g * 4 + u
                gdrain(u)
                wb(i, u)
                nxt = i + 2
                b2 = (u + 2) % 4

                @pl.when((nxt >= 4) & (nxt < nchunk))
                def _refill():
                    wdrain(b2)
                    start(nxt, b2)

            return carry

        lax.fori_loop(0, nchunk // 4, body, 0)
        for b in range(4):
            wdrain(b)

    return gather_kernel(table, idx3)


_NT = 8


def _make_gru_body(tseg):
    def _gru_body(e_ref, h0_ref, wih_ref, whh_ref, bih_ref, bhh_ref, fcw_ref,
                  fcb_ref, hout_ref, y_ref, h_scr):
        t = pl.program_id(0)

        @pl.when(t == 0)
        def _init():
            h_scr[...] = h0_ref[...]

        h = h_scr[...]
        for j in range(_NT):
            e = e_ref[j]
            gi = lax.dot_general(wih_ref[...], e, (((1,), (1,)), ((), ())),
                                 preferred_element_type=jnp.float32) + bih_ref[...]
            gh = jnp.dot(whh_ref[...], h, preferred_element_type=jnp.float32) + bhh_ref[...]
            r = jax.nn.sigmoid(gi[:HIDDEN] + gh[:HIDDEN])
            z = jax.nn.sigmoid(gi[HIDDEN:2 * HIDDEN] + gh[HIDDEN:2 * HIDDEN])
            n = jnp.tanh(gi[2 * HIDDEN:] + r * gh[2 * HIDDEN:])
            h = (1.0 - z) * n + z * h
        h_scr[...] = h

        @pl.when(t == tseg // _NT - 1)
        def _finish():
            hout_ref[...] = h
            logits = jnp.sum(h * fcw_ref[...], axis=0, keepdims=True) + fcb_ref[...]
            y_ref[...] = jax.nn.sigmoid(logits)

    return _gru_body


def _tc_gru_seg(e3, h0, wih, whh, bih2, bhh2, fcwT, fcb2, tseg):
    return pl.pallas_call(
        _make_gru_body(tseg),
        grid=(tseg // _NT,),
        in_specs=[
            pl.BlockSpec((_NT, B, EMBED), lambda t: (t, 0, 0)),
            pl.BlockSpec((HIDDEN, B), lambda t: (0, 0)),
            pl.BlockSpec((3 * HIDDEN, EMBED), lambda t: (0, 0)),
            pl.BlockSpec((3 * HIDDEN, HIDDEN), lambda t: (0, 0)),
            pl.BlockSpec((3 * HIDDEN, 1), lambda t: (0, 0)),
            pl.BlockSpec((3 * HIDDEN, 1), lambda t: (0, 0)),
            pl.BlockSpec((HIDDEN, 1), lambda t: (0, 0)),
            pl.BlockSpec((1, 1), lambda t: (0, 0)),
        ],
        out_specs=[
            pl.BlockSpec((HIDDEN, B), lambda t: (0, 0)),
            pl.BlockSpec((1, B), lambda t: (0, 0)),
        ],
        out_shape=[
            jax.ShapeDtypeStruct((HIDDEN, B), jnp.float32),
            jax.ShapeDtypeStruct((1, B), jnp.float32),
        ],
        scratch_shapes=[pltpu.VMEM((HIDDEN, B), jnp.float32)],
    )(e3, h0, wih, whh, bih2, bhh2, fcwT, fcb2)


def kernel(x, emb, W_ih, W_hh, b_ih, b_hh, fc_w, fc_b):
    idx = x.astype(jnp.int32).T.reshape(-1)

    bih2 = b_ih.reshape(-1, 1)
    bhh2 = b_hh.reshape(-1, 1)
    fcwT = fc_w.reshape(-1, 1)
    fcb2 = fc_b.reshape(1, 1)

    h = jnp.zeros((HIDDEN, B), jnp.float32)
    y = None
    off = 0
    for tseg in _SEGS:
        seg_rows = tseg * B
        nchunk = seg_rows // (_NW * _CHUNK)
        idx3 = lax.dynamic_slice(idx, (off * B,), (seg_rows,)).reshape(
            _NW, nchunk, _CHUNK)
        e = _sc_gather(emb, idx3, seg_rows, nchunk)
        e3 = e.reshape(tseg, B, EMBED)
        h, y = _tc_gru_seg(e3, h, W_ih, W_hh, bih2, bhh2, fcwT, fcb2, tseg)
        off += tseg
    return y.reshape(B, 1)


# speedup vs baseline: 1.0263x; 1.0014x over previous
import functools

import jax
import jax.numpy as jnp
from jax import lax
from jax.experimental import pallas as pl
from jax.experimental.pallas import tpu as pltpu
from jax.experimental.pallas import tpu_sc as plsc

EMBED = 128
HIDDEN = 64
B = 1024
T = 200

_SEGS = (16, 64, 64, 48, 8)

_NC = 2
_NS = 16
_NW = _NC * _NS
_CHUNK = 64
_NBUF = 8


def _sc_gather(table, idx3, rows, nchunk):
    rpw = rows // _NW
    mesh = plsc.VectorSubcoreMesh(core_axis_name="c", subcore_axis_name="s")

    @functools.partial(
        pl.kernel,
        mesh=mesh,
        out_type=jax.ShapeDtypeStruct((rows, EMBED), jnp.float32),
        scratch_types=(
            [pltpu.VMEM((nchunk, _CHUNK), jnp.int32)]
            + [pltpu.VMEM((_CHUNK, EMBED), jnp.float32)] * _NBUF
            + [pltpu.SemaphoreType.DMA] * (2 * _NBUF)
        ),
    )
    def gather_kernel(table_hbm, idx_hbm, out_hbm, idx_v, *rs):
        bufs = rs[:_NBUF]
        gsems = rs[_NBUF:2 * _NBUF]
        wsems = rs[2 * _NBUF:]
        wid = lax.axis_index("s") * _NC + lax.axis_index("c")
        base = wid * rpw
        pltpu.sync_copy(idx_hbm.at[wid], idx_v)

        def start(i, b):
            pltpu.async_copy(table_hbm.at[idx_v.at[i]], bufs[b], gsems[b])

        def gdrain(b):
            pltpu.make_async_copy(table_hbm.at[idx_v.at[0]], bufs[b], gsems[b]).wait()

        def wb(i, b):
            off = pl.multiple_of(base + i * _CHUNK, _CHUNK)
            pltpu.async_copy(bufs[b], out_hbm.at[pl.ds(off, _CHUNK)], wsems[b])

        def wdrain(b):
            pltpu.make_async_copy(bufs[b], out_hbm.at[pl.ds(0, _CHUNK)], wsems[b]).wait()

        if nchunk < _NBUF:
            for i in range(nchunk):
                start(i, i)
            for i in range(nchunk):
                gdrain(i)
                wb(i, i)
            for i in range(nchunk):
                wdrain(i)
            return

        half = _NBUF // 2
        for b in range(_NBUF):
            start(b, b)

        def body(g, carry):
            for u in range(_NBUF):
                i = g * _NBUF + u
                gdrain(u)
                wb(i, u)
                nxt = i + half
                b2 = (u + half) % _NBUF

                @pl.when((nxt >= _NBUF) & (nxt < nchunk))
                def _refill():
                    wdrain(b2)
                    start(nxt, b2)

            return carry

        lax.fori_loop(0, nchunk // _NBUF, body, 0)
        for b in range(_NBUF):
            wdrain(b)

    return gather_kernel(table, idx3)


_NT = 8


def _make_gru_body(tseg):
    def _gru_body(e_ref, h0_ref, wih_ref, whh_ref, bih_ref, bhh_ref, fcw_ref,
                  fcb_ref, hout_ref, y_ref, h_scr):
        t = pl.program_id(0)

        @pl.when(t == 0)
        def _init():
            h_scr[...] = h0_ref[...]

        h = h_scr[...]
        for j in range(_NT):
            e = e_ref[j]
            gi = lax.dot_general(wih_ref[...], e, (((1,), (1,)), ((), ())),
                                 preferred_element_type=jnp.float32) + bih_ref[...]
            gh = jnp.dot(whh_ref[...], h, preferred_element_type=jnp.float32) + bhh_ref[...]
            r = jax.nn.sigmoid(gi[:HIDDEN] + gh[:HIDDEN])
            z = jax.nn.sigmoid(gi[HIDDEN:2 * HIDDEN] + gh[HIDDEN:2 * HIDDEN])
            n = jnp.tanh(gi[2 * HIDDEN:] + r * gh[2 * HIDDEN:])
            h = (1.0 - z) * n + z * h
        h_scr[...] = h

        @pl.when(t == tseg // _NT - 1)
        def _finish():
            hout_ref[...] = h
            logits = jnp.sum(h * fcw_ref[...], axis=0, keepdims=True) + fcb_ref[...]
            y_ref[...] = jax.nn.sigmoid(logits)

    return _gru_body


def _tc_gru_seg(e3, h0, wih, whh, bih2, bhh2, fcwT, fcb2, tseg):
    return pl.pallas_call(
        _make_gru_body(tseg),
        grid=(tseg // _NT,),
        in_specs=[
            pl.BlockSpec((_NT, B, EMBED), lambda t: (t, 0, 0)),
            pl.BlockSpec((HIDDEN, B), lambda t: (0, 0)),
            pl.BlockSpec((3 * HIDDEN, EMBED), lambda t: (0, 0)),
            pl.BlockSpec((3 * HIDDEN, HIDDEN), lambda t: (0, 0)),
            pl.BlockSpec((3 * HIDDEN, 1), lambda t: (0, 0)),
            pl.BlockSpec((3 * HIDDEN, 1), lambda t: (0, 0)),
            pl.BlockSpec((HIDDEN, 1), lambda t: (0, 0)),
            pl.BlockSpec((1, 1), lambda t: (0, 0)),
        ],
        out_specs=[
            pl.BlockSpec((HIDDEN, B), lambda t: (0, 0)),
            pl.BlockSpec((1, B), lambda t: (0, 0)),
        ],
        out_shape=[
            jax.ShapeDtypeStruct((HIDDEN, B), jnp.float32),
            jax.ShapeDtypeStruct((1, B), jnp.float32),
        ],
        scratch_shapes=[pltpu.VMEM((HIDDEN, B), jnp.float32)],
    )(e3, h0, wih, whh, bih2, bhh2, fcwT, fcb2)


def kernel(x, emb, W_ih, W_hh, b_ih, b_hh, fc_w, fc_b):
    idx = x.astype(jnp.int32).T.reshape(-1)

    bih2 = b_ih.reshape(-1, 1)
    bhh2 = b_hh.reshape(-1, 1)
    fcwT = fc_w.reshape(-1, 1)
    fcb2 = fc_b.reshape(1, 1)

    h = jnp.zeros((HIDDEN, B), jnp.float32)
    y = None
    off = 0
    for tseg in _SEGS:
        seg_rows = tseg * B
        nchunk = seg_rows // (_NW * _CHUNK)
        idx3 = lax.dynamic_slice(idx, (off * B,), (seg_rows,)).reshape(
            _NW, nchunk, _CHUNK)
        e = _sc_gather(emb, idx3, seg_rows, nchunk)
        e3 = e.reshape(tseg, B, EMBED)
        h, y = _tc_gru_seg(e3, h, W_ih, W_hh, bih2, bhh2, fcwT, fcb2, tseg)
        off += tseg
    return y.reshape(B, 1)


# speedup vs baseline: 1.0416x; 1.0149x over previous
import functools

import jax
import jax.numpy as jnp
from jax import lax
from jax.experimental import pallas as pl
from jax.experimental.pallas import tpu as pltpu
from jax.experimental.pallas import tpu_sc as plsc

EMBED = 128
HIDDEN = 64
B = 1024
T = 200

_SEGS = (16, 64, 64, 48, 8)

_NC = 2
_NS = 16
_NW = _NC * _NS
_CHUNK = 64
_NBUF = 8


def _sc_gather(table, idx_full, rows, nchunk, seg_off):
    rpw = rows // _NW
    mesh = plsc.VectorSubcoreMesh(core_axis_name="c", subcore_axis_name="s")

    @functools.partial(
        pl.kernel,
        mesh=mesh,
        out_type=jax.ShapeDtypeStruct((rows, EMBED), jnp.float32),
        scratch_types=(
            [pltpu.VMEM((rpw,), jnp.int32)]
            + [pltpu.VMEM((_CHUNK, EMBED), jnp.float32)] * _NBUF
            + [pltpu.SemaphoreType.DMA] * (2 * _NBUF)
        ),
    )
    def gather_kernel(table_hbm, idx_hbm, out_hbm, idx_v, *rs):
        bufs = rs[:_NBUF]
        gsems = rs[_NBUF:2 * _NBUF]
        wsems = rs[2 * _NBUF:]
        wid = lax.axis_index("s") * _NC + lax.axis_index("c")
        base = wid * rpw
        src_off = pl.multiple_of(seg_off + base, _CHUNK)
        pltpu.sync_copy(idx_hbm.at[pl.ds(src_off, rpw)], idx_v)

        def start(i, b):
            sl = pl.ds(pl.multiple_of(i * _CHUNK, _CHUNK), _CHUNK)
            pltpu.async_copy(table_hbm.at[idx_v.at[sl]], bufs[b], gsems[b])

        def gdrain(b):
            pltpu.make_async_copy(
                table_hbm.at[idx_v.at[pl.ds(0, _CHUNK)]], bufs[b], gsems[b]).wait()

        def wb(i, b):
            off = pl.multiple_of(base + i * _CHUNK, _CHUNK)
            pltpu.async_copy(bufs[b], out_hbm.at[pl.ds(off, _CHUNK)], wsems[b])

        def wdrain(b):
            pltpu.make_async_copy(bufs[b], out_hbm.at[pl.ds(0, _CHUNK)], wsems[b]).wait()

        if nchunk < _NBUF:
            for i in range(nchunk):
                start(i, i)
            for i in range(nchunk):
                gdrain(i)
                wb(i, i)
            for i in range(nchunk):
                wdrain(i)
            return

        half = _NBUF // 2
        for b in range(_NBUF):
            start(b, b)

        def body(g, carry):
            for u in range(_NBUF):
                i = g * _NBUF + u
                gdrain(u)
                wb(i, u)
                nxt = i + half
                b2 = (u + half) % _NBUF

                @pl.when((nxt >= _NBUF) & (nxt < nchunk))
                def _refill():
                    wdrain(b2)
                    start(nxt, b2)

            return carry

        lax.fori_loop(0, nchunk // _NBUF, body, 0)
        for b in range(_NBUF):
            wdrain(b)

    return gather_kernel(table, idx_full)


_NT = 8


def _make_gru_body(tseg):
    def _gru_body(e_ref, h0_ref, wih_ref, whh_ref, bih_ref, bhh_ref, fcw_ref,
                  fcb_ref, hout_ref, y_ref, h_scr):
        t = pl.program_id(0)

        @pl.when(t == 0)
        def _init():
            h_scr[...] = h0_ref[...]

        h = h_scr[...]
        for j in range(_NT):
            e = e_ref[j]
            gi = lax.dot_general(wih_ref[...], e, (((1,), (1,)), ((), ())),
                                 preferred_element_type=jnp.float32) + bih_ref[...]
            gh = jnp.dot(whh_ref[...], h, preferred_element_type=jnp.float32) + bhh_ref[...]
            r = jax.nn.sigmoid(gi[:HIDDEN] + gh[:HIDDEN])
            z = jax.nn.sigmoid(gi[HIDDEN:2 * HIDDEN] + gh[HIDDEN:2 * HIDDEN])
            n = jnp.tanh(gi[2 * HIDDEN:] + r * gh[2 * HIDDEN:])
            h = (1.0 - z) * n + z * h
        h_scr[...] = h

        @pl.when(t == tseg // _NT - 1)
        def _finish():
            hout_ref[...] = h
            logits = jnp.sum(h * fcw_ref[...], axis=0, keepdims=True) + fcb_ref[...]
            y_ref[...] = jax.nn.sigmoid(logits)

    return _gru_body


def _tc_gru_seg(e3, h0, wih, whh, bih2, bhh2, fcwT, fcb2, tseg):
    return pl.pallas_call(
        _make_gru_body(tseg),
        grid=(tseg // _NT,),
        in_specs=[
            pl.BlockSpec((_NT, B, EMBED), lambda t: (t, 0, 0)),
            pl.BlockSpec((HIDDEN, B), lambda t: (0, 0)),
            pl.BlockSpec((3 * HIDDEN, EMBED), lambda t: (0, 0)),
            pl.BlockSpec((3 * HIDDEN, HIDDEN), lambda t: (0, 0)),
            pl.BlockSpec((3 * HIDDEN, 1), lambda t: (0, 0)),
            pl.BlockSpec((3 * HIDDEN, 1), lambda t: (0, 0)),
            pl.BlockSpec((HIDDEN, 1), lambda t: (0, 0)),
            pl.BlockSpec((1, 1), lambda t: (0, 0)),
        ],
        out_specs=[
            pl.BlockSpec((HIDDEN, B), lambda t: (0, 0)),
            pl.BlockSpec((1, B), lambda t: (0, 0)),
        ],
        out_shape=[
            jax.ShapeDtypeStruct((HIDDEN, B), jnp.float32),
            jax.ShapeDtypeStruct((1, B), jnp.float32),
        ],
        scratch_shapes=[pltpu.VMEM((HIDDEN, B), jnp.float32)],
    )(e3, h0, wih, whh, bih2, bhh2, fcwT, fcb2)


def kernel(x, emb, W_ih, W_hh, b_ih, b_hh, fc_w, fc_b):
    idx = x.astype(jnp.int32).T.reshape(-1)

    bih2 = b_ih.reshape(-1, 1)
    bhh2 = b_hh.reshape(-1, 1)
    fcwT = fc_w.reshape(-1, 1)
    fcb2 = fc_b.reshape(1, 1)

    h = jnp.zeros((HIDDEN, B), jnp.float32)
    y = None
    off = 0
    for tseg in _SEGS:
        seg_rows = tseg * B
        nchunk = seg_rows // (_NW * _CHUNK)
        e = _sc_gather(emb, idx, seg_rows, nchunk, off * B)
        e3 = e.reshape(tseg, B, EMBED)
        h, y = _tc_gru_seg(e3, h, W_ih, W_hh, bih2, bhh2, fcwT, fcb2, tseg)
        off += tseg
    return y.reshape(B, 1)
